# baseline (speedup 1.0000x reference)
import jax
import jax.numpy as jnp
from jax import lax
from jax.experimental import pallas as pl
from jax.experimental.pallas import tpu as pltpu
from jax.experimental.pallas import tpu_sc as plsc

B, S, H = 4, 2048, 768
NC, NS = 2, 16
NW = NC * NS
PPW = S // NW
EPS = 1e-12

BLK = 2048
SBLK = S // BLK


def _sc_gather_body(ids_hbm, word_hbm, out_hbm,
                    idsbuf, buf0, buf1, semg0, semg1, semo0, semo1):
    wid = lax.axis_index("s") * NC + lax.axis_index("c")
    p0 = wid * PPW
    bufs = (buf0, buf1)
    semgs = (semg0, semg1)
    semos = (semo0, semo1)

    def stage(b, slot):
        pltpu.sync_copy(ids_hbm.at[b, pl.ds(p0, PPW)], idsbuf.at[slot])

    def gather(slot):
        return pltpu.async_copy(word_hbm.at[idsbuf.at[slot]],
                                bufs[slot], semgs[slot])

    stage(0, 0)
    gathers = [gather(0), None]
    outs = [None, None]
    for b in range(B):
        buf = b & 1
        nbuf = 1 - buf
        if b + 1 < B:
            stage(b + 1, nbuf)
            if outs[nbuf] is not None:
                outs[nbuf].wait()
            gathers[nbuf] = gather(nbuf)
        gathers[buf].wait()
        outs[buf] = pltpu.async_copy(
            bufs[buf], out_hbm.at[b, pl.ds(p0, PPW)], semos[buf])
    for slot in range(2):
        if outs[slot] is not None:
            outs[slot].wait()


def _tc_ln_body(g_ref, pos_ref, tid_ref, t_ref, gam_ref, bet_ref, o_ref):
    x = g_ref[...]
    tid = tid_ref[...]
    x = x + pos_ref[...] + jnp.where(tid == 0, t_ref[0:1, :], t_ref[1:2, :])
    mean = jnp.mean(x, axis=-1, keepdims=True)
    xc = x - mean
    var = jnp.mean(xc * xc, axis=-1, keepdims=True)
    o_ref[...] = xc * lax.rsqrt(var + EPS) * gam_ref[...] + bet_ref[...]


def kernel(input_ids, token_type_ids, word_emb, pos_emb, type_emb, gamma, beta):
    mesh = plsc.VectorSubcoreMesh(core_axis_name="c", subcore_axis_name="s",
                                  num_cores=NC, num_subcores=NS)
    sc_gather = pl.kernel(
        _sc_gather_body,
        out_type=jax.ShapeDtypeStruct((B, S, H), jnp.float32),
        mesh=mesh,
        compiler_params=pltpu.CompilerParams(needs_layout_passes=False),
        scratch_types=[
            pltpu.VMEM((2, PPW), jnp.int32),
            pltpu.VMEM((PPW, H), jnp.float32),
            pltpu.VMEM((PPW, H), jnp.float32),
            pltpu.SemaphoreType.DMA,
            pltpu.SemaphoreType.DMA,
            pltpu.SemaphoreType.DMA,
            pltpu.SemaphoreType.DMA,
        ],
    )
    gathered = sc_gather(input_ids, word_emb)

    ln = pl.pallas_call(
        _tc_ln_body,
        grid=(SBLK, B),
        in_specs=[
            pl.BlockSpec((BLK, H), lambda i, j: (j * SBLK + i, 0)),
            pl.BlockSpec((BLK, H), lambda i, j: (i, 0)),
            pl.BlockSpec((BLK, 1), lambda i, j: (j * SBLK + i, 0)),
            pl.BlockSpec((2, H), lambda i, j: (0, 0)),
            pl.BlockSpec((1, H), lambda i, j: (0, 0)),
            pl.BlockSpec((1, H), lambda i, j: (0, 0)),
        ],
        out_specs=pl.BlockSpec((BLK, H), lambda i, j: (j * SBLK + i, 0)),
        out_shape=jax.ShapeDtypeStruct((B * S, H), jnp.float32),
    )
    out = ln(gathered.reshape(B * S, H), pos_emb,
             token_type_ids.reshape(B * S, 1), type_emb,
             gamma.reshape(1, H), beta.reshape(1, H))
    return out.reshape(B, S, H)


# speedup vs baseline: 1.0071x; 1.0071x over previous
import jax
import jax.numpy as jnp
from jax import lax
from jax.experimental import pallas as pl
from jax.experimental.pallas import tpu as pltpu
from jax.experimental.pallas import tpu_sc as plsc

B, S, H = 4, 2048, 768
BH = B // 2
NC, NS = 2, 16
NW = NC * NS
PPW = S // NW
EPS = 1e-12

BLK = 2048


def _sc_gather_body(ids_hbm, word_hbm, out_hbm,
                    idsbuf, buf0, buf1, semg0, semg1, semo0, semo1):
    wid = lax.axis_index("s") * NC + lax.axis_index("c")
    p0 = wid * PPW
    bufs = (buf0, buf1)
    semgs = (semg0, semg1)
    semos = (semo0, semo1)

    def stage(b, slot):
        pltpu.sync_copy(ids_hbm.at[b, pl.ds(p0, PPW)], idsbuf.at[slot])

    def gather(slot):
        return pltpu.async_copy(word_hbm.at[idsbuf.at[slot]],
                                bufs[slot], semgs[slot])

    stage(0, 0)
    gathers = [gather(0), None]
    outs = [None, None]
    for b in range(BH):
        buf = b & 1
        nbuf = 1 - buf
        if b + 1 < BH:
            stage(b + 1, nbuf)
            if outs[nbuf] is not None:
                outs[nbuf].wait()
            gathers[nbuf] = gather(nbuf)
        gathers[buf].wait()
        outs[buf] = pltpu.async_copy(
            bufs[buf], out_hbm.at[b, pl.ds(p0, PPW)], semos[buf])
    for slot in range(2):
        if outs[slot] is not None:
            outs[slot].wait()


def _tc_ln_body(g_ref, pos_ref, tid_ref, t_ref, gam_ref, bet_ref, o_ref):
    x = g_ref[...]
    tid = tid_ref[...]
    x = x + pos_ref[...] + jnp.where(tid == 0, t_ref[0:1, :], t_ref[1:2, :])
    mean = jnp.mean(x, axis=-1, keepdims=True)
    xc = x - mean
    var = jnp.mean(xc * xc, axis=-1, keepdims=True)
    o_ref[...] = xc * lax.rsqrt(var + EPS) * gam_ref[...] + bet_ref[...]


def _tc_ln_body_alias(g_ref, pos_ref, tid_ref, t_ref, gam_ref, bet_ref,
                      prev_ref, o_ref):
    del prev_ref
    _tc_ln_body(g_ref, pos_ref, tid_ref, t_ref, gam_ref, bet_ref, o_ref)


def kernel(input_ids, token_type_ids, word_emb, pos_emb, type_emb, gamma, beta):
    mesh = plsc.VectorSubcoreMesh(core_axis_name="c", subcore_axis_name="s",
                                  num_cores=NC, num_subcores=NS)
    sc_gather = pl.kernel(
        _sc_gather_body,
        out_type=jax.ShapeDtypeStruct((BH, S, H), jnp.float32),
        mesh=mesh,
        compiler_params=pltpu.CompilerParams(needs_layout_passes=False),
        scratch_types=[
            pltpu.VMEM((2, PPW), jnp.int32),
            pltpu.VMEM((PPW, H), jnp.float32),
            pltpu.VMEM((PPW, H), jnp.float32),
            pltpu.SemaphoreType.DMA,
            pltpu.SemaphoreType.DMA,
            pltpu.SemaphoreType.DMA,
            pltpu.SemaphoreType.DMA,
        ],
    )
    g0 = sc_gather(input_ids[:BH], word_emb)
    g1 = sc_gather(input_ids[BH:], word_emb)

    gam2 = gamma.reshape(1, H)
    bet2 = beta.reshape(1, H)
    tid0 = token_type_ids[:BH].reshape(BH * S, 1)
    tid1 = token_type_ids[BH:].reshape(BH * S, 1)

    common = dict(
        grid=(1, BH),
        in_specs=[
            pl.BlockSpec((BLK, H), lambda i, j: (j, 0)),
            pl.BlockSpec((BLK, H), lambda i, j: (i, 0)),
            pl.BlockSpec((BLK, 1), lambda i, j: (j, 0)),
            pl.BlockSpec((2, H), lambda i, j: (0, 0)),
            pl.BlockSpec((1, H), lambda i, j: (0, 0)),
            pl.BlockSpec((1, H), lambda i, j: (0, 0)),
        ],
        out_shape=jax.ShapeDtypeStruct((B * S, H), jnp.float32),
    )
    out0 = pl.pallas_call(
        _tc_ln_body,
        out_specs=pl.BlockSpec((BLK, H), lambda i, j: (j, 0)),
        **common,
    )(g0.reshape(BH * S, H), pos_emb, tid0, type_emb, gam2, bet2)

    common["in_specs"] = common["in_specs"] + [
        pl.BlockSpec(memory_space=pl.ANY)]
    out = pl.pallas_call(
        _tc_ln_body_alias,
        out_specs=pl.BlockSpec((BLK, H), lambda i, j: (j + BH, 0)),
        input_output_aliases={6: 0},
        **common,
    )(g1.reshape(BH * S, H), pos_emb, tid1, type_emb, gam2, bet2, out0)
    return out.reshape(B, S, H)
